# Initial kernel scaffold; baseline (speedup 1.0000x reference)
import jax
import jax.numpy as jnp
from jax.experimental import pallas as pl


def kernel(x, edge_index, W, bconv, eps):
    raise NotImplementedError("write your pallas kernel here")


# speedup vs baseline: 2.6885x; 2.6885x over previous
import functools

import jax
import jax.numpy as jnp
from jax import lax
from jax.experimental import pallas as pl
from jax.experimental.pallas import tpu as pltpu
from jax.experimental.pallas import tpu_sc as plsc

B, C_IN, C_OUT, N, K = 4, 256, 256, 4096, 16
ROWS = B * N
NW = 32
RPW = ROWS // NW
G = 8
LANES = 16
NB = 1024


def _sc_body(xt, idxg, eps16, h, idx_v, rows_v, own_v, acc_v, eps_v, sem):
    wid = lax.axis_index("s") * 2 + lax.axis_index("c")
    base = wid * RPW
    pltpu.sync_copy(eps16, eps_v)
    e = eps_v[...]

    def blk(i, carry):
        nb = base + i * G
        pltpu.sync_copy(idxg.at[pl.ds(nb * K, G * K)], idx_v)
        gth = pltpu.async_copy(xt.at[idx_v], rows_v, sem)
        pltpu.sync_copy(xt.at[pl.ds(nb, G)], own_v)
        gth.wait()

        def node(g, carry2):
            for cb in range(C_IN // LANES):
                sl = pl.ds(cb * LANES, LANES)
                acc = own_v[g, sl] * e
                for k in range(K):
                    acc = acc + rows_v[g * K + k, sl]
                acc_v[g, sl] = acc
            return carry2

        lax.fori_loop(0, G, node, 0)
        pltpu.sync_copy(acc_v, h.at[pl.ds(nb, G)])
        return carry

    lax.fori_loop(0, RPW // G, blk, 0)


_sc_gather = functools.partial(
    pl.kernel,
    mesh=plsc.VectorSubcoreMesh(core_axis_name="c", subcore_axis_name="s"),
    out_type=jax.ShapeDtypeStruct((ROWS, C_IN), jnp.float32),
    scratch_types=[
        pltpu.VMEM((G * K,), jnp.int32),
        pltpu.VMEM((G * K, C_IN), jnp.float32),
        pltpu.VMEM((G, C_IN), jnp.float32),
        pltpu.VMEM((G, C_IN), jnp.float32),
        pltpu.VMEM((LANES,), jnp.float32),
        pltpu.SemaphoreType.DMA,
    ],
)(_sc_body)


def _tc_body(h_ref, w_ref, b_ref, o_ref):
    hb = h_ref[...]
    w = w_ref[...]
    out = lax.dot_general(w, hb, (((1,), (1,)), ((), ())),
                          preferred_element_type=jnp.float32)
    o_ref[...] = jnp.maximum(out + b_ref[...], 0.0)[None]


def _tc_conv(h, W, b2):
    return pl.pallas_call(
        _tc_body,
        grid=(ROWS // NB,),
        in_specs=[
            pl.BlockSpec((NB, C_IN), lambda i: (i, 0)),
            pl.BlockSpec((C_OUT, C_IN), lambda i: (0, 0)),
            pl.BlockSpec((C_OUT, 1), lambda i: (0, 0)),
        ],
        out_specs=pl.BlockSpec((1, C_OUT, NB),
                               lambda i: (i // (N // NB), 0, i % (N // NB))),
        out_shape=jax.ShapeDtypeStruct((B, C_OUT, N), jnp.float32),
    )(h, W, b2)


def kernel(x, edge_index, W, bconv, eps):
    xt = x[:, :, :, 0].transpose(0, 2, 1).reshape(ROWS, C_IN)
    idxg = (edge_index[0]
            + (jnp.arange(B, dtype=jnp.int32) * N)[:, None, None]).reshape(-1)
    eps16 = jnp.full((LANES,), 1.0, jnp.float32) + eps
    h = _sc_gather(xt, idxg, eps16)
    out = _tc_conv(h, W, bconv.reshape(C_OUT, 1))
    return out.reshape(B, C_OUT, N, 1)


# speedup vs baseline: 4.1010x; 1.5254x over previous
import functools

import jax
import jax.numpy as jnp
from jax import lax
from jax.experimental import pallas as pl
from jax.experimental.pallas import tpu as pltpu
from jax.experimental.pallas import tpu_sc as plsc

B, C_IN, C_OUT, N, K = 4, 256, 256, 4096, 16
ROWS = B * N
NW = 32
RPW = ROWS // NW
G = 8
NBLK = RPW // G
NBUF = 2
LANES = 16
NB = 1024


def _sc_body(xt, idxg, xj, idx_all, rows0, rows1, acc0, acc1,
             sg0, sg1, so0, so1):
    rows_v = (rows0, rows1)
    acc_v = (acc0, acc1)
    sem_g = (sg0, sg1)
    sem_o = (so0, so1)
    wid = lax.axis_index("s") * 2 + lax.axis_index("c")
    base = wid * RPW

    pltpu.sync_copy(idxg.at[pl.ds(base * K, RPW * K)], idx_all)

    def start_gather(i, b):
        pltpu.async_copy(
            xt.at[idx_all.at[pl.ds(i * G * K, G * K)]], rows_v[b], sem_g[b])

    def out_slice(i):
        return xj.at[pl.ds(base + i * G, G)]

    for b in range(NBUF):
        start_gather(b, b)

    def do_block(i, b):
        @pl.when(i >= NBUF)
        def _():
            pltpu.make_async_copy(acc_v[b], out_slice(i - NBUF),
                                  sem_o[b]).wait()

        pltpu.make_async_copy(xt.at[idx_all.at[pl.ds(i * G * K, G * K)]],
                              rows_v[b], sem_g[b]).wait()

        def node(g, carry):
            for cb in range(C_IN // LANES):
                sl = pl.ds(cb * LANES, LANES)
                acc = rows_v[b][g * K, sl]
                for k in range(1, K):
                    acc = acc + rows_v[b][g * K + k, sl]
                acc_v[b][g, sl] = acc
            return carry

        lax.fori_loop(0, G, node, 0)
        pltpu.async_copy(acc_v[b], out_slice(i), sem_o[b])

        @pl.when(i + NBUF < NBLK)
        def _():
            start_gather(i + NBUF, b)

    def blk(j, carry):
        for b in range(NBUF):
            do_block(j * NBUF + b, b)
        return carry

    lax.fori_loop(0, NBLK // NBUF, blk, 0)
    for b in range(NBUF):
        pltpu.make_async_copy(acc_v[b], out_slice(NBLK - NBUF + b),
                              sem_o[b]).wait()


_sc_gather = functools.partial(
    pl.kernel,
    mesh=plsc.VectorSubcoreMesh(core_axis_name="c", subcore_axis_name="s"),
    out_type=jax.ShapeDtypeStruct((ROWS, C_IN), jnp.float32),
    scratch_types=[
        pltpu.VMEM((RPW * K,), jnp.int32),
        pltpu.VMEM((G * K, C_IN), jnp.float32),
        pltpu.VMEM((G * K, C_IN), jnp.float32),
        pltpu.VMEM((G, C_IN), jnp.float32),
        pltpu.VMEM((G, C_IN), jnp.float32),
        pltpu.SemaphoreType.DMA,
        pltpu.SemaphoreType.DMA,
        pltpu.SemaphoreType.DMA,
        pltpu.SemaphoreType.DMA,
    ],
)(_sc_body)


def _tc_body(x_ref, xj_ref, we_ref, w_ref, b_ref, o_ref):
    xb = x_ref[0]
    hj = xj_ref[...]
    s1 = lax.dot_general(we_ref[...], xb, (((1,), (0,)), ((), ())),
                         preferred_element_type=jnp.float32)
    s2 = lax.dot_general(w_ref[...], hj, (((1,), (1,)), ((), ())),
                         preferred_element_type=jnp.float32)
    o_ref[...] = jnp.maximum(s1 + s2 + b_ref[...], 0.0)[None]


def _tc_conv(x3, xj, W_eps, W, b2):
    nblk = N // NB
    return pl.pallas_call(
        _tc_body,
        grid=(ROWS // NB,),
        in_specs=[
            pl.BlockSpec((1, C_IN, NB), lambda i: (i // nblk, 0, i % nblk)),
            pl.BlockSpec((NB, C_IN), lambda i: (i, 0)),
            pl.BlockSpec((C_OUT, C_IN), lambda i: (0, 0)),
            pl.BlockSpec((C_OUT, C_IN), lambda i: (0, 0)),
            pl.BlockSpec((C_OUT, 1), lambda i: (0, 0)),
        ],
        out_specs=pl.BlockSpec((1, C_OUT, NB),
                               lambda i: (i // nblk, 0, i % nblk)),
        out_shape=jax.ShapeDtypeStruct((B, C_OUT, N), jnp.float32),
    )(x3, xj, W_eps, W, b2)


def kernel(x, edge_index, W, bconv, eps):
    xt = x[:, :, :, 0].transpose(0, 2, 1).reshape(ROWS, C_IN)
    idxg = (edge_index[0]
            + (jnp.arange(B, dtype=jnp.int32) * N)[:, None, None]).reshape(-1)
    xj = _sc_gather(xt, idxg)
    W_eps = (1.0 + eps[0]) * W
    out = _tc_conv(x.reshape(B, C_IN, N), xj, W_eps, W,
                   bconv.reshape(C_OUT, 1))
    return out.reshape(B, C_OUT, N, 1)


# speedup vs baseline: 4.1985x; 1.0238x over previous
import functools

import jax
import jax.numpy as jnp
from jax import lax
from jax.experimental import pallas as pl
from jax.experimental.pallas import tpu as pltpu
from jax.experimental.pallas import tpu_sc as plsc

B, C_IN, C_OUT, N, K = 4, 256, 256, 4096, 16
ROWS = B * N
NW = 32
RPW = ROWS // NW
G = 8
NBLK = RPW // G
NBUF = 2
LANES = 16
NB = 1024


def _sc_body(xt, idxg, xj, idx_all, rows0, rows1, acc0, acc1,
             sg0, sg1, so0, so1):
    rows_v = (rows0, rows1)
    acc_v = (acc0, acc1)
    sem_g = (sg0, sg1)
    sem_o = (so0, so1)
    wid = lax.axis_index("s") * 2 + lax.axis_index("c")
    base = wid * RPW

    pltpu.sync_copy(idxg.at[pl.ds(base * K, RPW * K)], idx_all)

    boff = lax.broadcast((wid // (NW // B)) * N, (LANES,))

    def rebase(i, carry):
        sl = pl.ds(i * LANES, LANES)
        idx_all[sl] = idx_all[sl] + boff
        return carry

    lax.fori_loop(0, RPW * K // LANES, rebase, 0)

    def start_gather(i, b):
        pltpu.async_copy(
            xt.at[idx_all.at[pl.ds(i * G * K, G * K)]], rows_v[b], sem_g[b])

    def out_slice(i):
        return xj.at[pl.ds(base + i * G, G)]

    for b in range(NBUF):
        start_gather(b, b)

    def do_block(i, b):
        @pl.when(i >= NBUF)
        def _():
            pltpu.make_async_copy(acc_v[b], out_slice(i - NBUF),
                                  sem_o[b]).wait()

        pltpu.make_async_copy(xt.at[idx_all.at[pl.ds(i * G * K, G * K)]],
                              rows_v[b], sem_g[b]).wait()

        def node(g, carry):
            for cb in range(C_IN // LANES):
                sl = pl.ds(cb * LANES, LANES)
                acc = rows_v[b][g * K, sl]
                for k in range(1, K):
                    acc = acc + rows_v[b][g * K + k, sl]
                acc_v[b][g, sl] = acc
            return carry

        lax.fori_loop(0, G, node, 0)
        pltpu.async_copy(acc_v[b], out_slice(i), sem_o[b])

        @pl.when(i + NBUF < NBLK)
        def _():
            start_gather(i + NBUF, b)

    def blk(j, carry):
        for b in range(NBUF):
            do_block(j * NBUF + b, b)
        return carry

    lax.fori_loop(0, NBLK // NBUF, blk, 0)
    for b in range(NBUF):
        pltpu.make_async_copy(acc_v[b], out_slice(NBLK - NBUF + b),
                              sem_o[b]).wait()


_sc_gather = functools.partial(
    pl.kernel,
    mesh=plsc.VectorSubcoreMesh(core_axis_name="c", subcore_axis_name="s"),
    out_type=jax.ShapeDtypeStruct((ROWS, C_IN), jnp.float32),
    scratch_types=[
        pltpu.VMEM((RPW * K,), jnp.int32),
        pltpu.VMEM((G * K, C_IN), jnp.float32),
        pltpu.VMEM((G * K, C_IN), jnp.float32),
        pltpu.VMEM((G, C_IN), jnp.float32),
        pltpu.VMEM((G, C_IN), jnp.float32),
        pltpu.SemaphoreType.DMA,
        pltpu.SemaphoreType.DMA,
        pltpu.SemaphoreType.DMA,
        pltpu.SemaphoreType.DMA,
    ],
)(_sc_body)


NBT = 2048


def _tr_body(x_ref, o_ref):
    o_ref[...] = x_ref[0].T


def _tc_transpose(x3):
    return pl.pallas_call(
        _tr_body,
        grid=(N // NBT, B),
        in_specs=[pl.BlockSpec((1, C_IN, NBT), lambda i, b: (b, 0, i))],
        out_specs=pl.BlockSpec((NBT, C_IN),
                               lambda i, b: (b * (N // NBT) + i, 0)),
        out_shape=jax.ShapeDtypeStruct((ROWS, C_IN), jnp.float32),
    )(x3)


def _tc_body(x_ref, xj_ref, we_ref, w_ref, b_ref, o_ref):
    xb = x_ref[0]
    hj = xj_ref[...]
    s1 = lax.dot_general(we_ref[...], xb, (((1,), (0,)), ((), ())),
                         preferred_element_type=jnp.float32)
    s2 = lax.dot_general(w_ref[...], hj, (((1,), (1,)), ((), ())),
                         preferred_element_type=jnp.float32)
    o_ref[...] = jnp.maximum(s1 + s2 + b_ref[...], 0.0)[None]


def _tc_conv(x3, xj, W_eps, W, b2):
    nblk = N // NB
    return pl.pallas_call(
        _tc_body,
        grid=(ROWS // NB,),
        in_specs=[
            pl.BlockSpec((1, C_IN, NB), lambda i: (i // nblk, 0, i % nblk)),
            pl.BlockSpec((NB, C_IN), lambda i: (i, 0)),
            pl.BlockSpec((C_OUT, C_IN), lambda i: (0, 0)),
            pl.BlockSpec((C_OUT, C_IN), lambda i: (0, 0)),
            pl.BlockSpec((C_OUT, 1), lambda i: (0, 0)),
        ],
        out_specs=pl.BlockSpec((1, C_OUT, NB),
                               lambda i: (i // nblk, 0, i % nblk)),
        out_shape=jax.ShapeDtypeStruct((B, C_OUT, N), jnp.float32),
    )(x3, xj, W_eps, W, b2)


def kernel(x, edge_index, W, bconv, eps):
    xt = _tc_transpose(x.reshape(B, C_IN, N))
    idxg = edge_index[0].reshape(-1)
    xj = _sc_gather(xt, idxg)
    W_eps = (1.0 + eps[0]) * W
    out = _tc_conv(x.reshape(B, C_IN, N), xj, W_eps, W,
                   bconv.reshape(C_OUT, 1))
    return out.reshape(B, C_OUT, N, 1)


# speedup vs baseline: 6.1676x; 1.4690x over previous
import functools

import jax
import jax.numpy as jnp
from jax import lax
from jax.experimental import pallas as pl
from jax.experimental.pallas import tpu as pltpu
from jax.experimental.pallas import tpu_sc as plsc

B, C_IN, C_OUT, N, K = 4, 256, 256, 4096, 16
ROWS = B * N
NW = 32
RPW = ROWS // NW
G = 8
NBLK = RPW // G
NBUF = 2
LANES = 16
NB = 1024


def _sc_body(xt, idxg, xj, idx_all, rows0, rows1, acc0, acc1,
             sg0, sg1, so0, so1):
    rows_v = (rows0, rows1)
    acc_v = (acc0, acc1)
    sem_g = (sg0, sg1)
    sem_o = (so0, so1)
    wid = lax.axis_index("s") * 2 + lax.axis_index("c")
    base = wid * RPW

    pltpu.sync_copy(idxg.at[pl.ds(base * K, RPW * K)], idx_all)

    boff = lax.broadcast((wid // (NW // B)) * N, (LANES,))

    def rebase(i, carry):
        sl = pl.ds(i * LANES, LANES)
        idx_all[sl] = idx_all[sl] + boff
        return carry

    lax.fori_loop(0, RPW * K // LANES, rebase, 0)

    def start_gather(i, b):
        pltpu.async_copy(
            xt.at[idx_all.at[pl.ds(i * G * K, G * K)]], rows_v[b], sem_g[b])

    def out_slice(i):
        return xj.at[pl.ds(base + i * G, G)]

    for b in range(NBUF):
        start_gather(b, b)

    def do_block(i, b):
        @pl.when(i >= NBUF)
        def _():
            pltpu.make_async_copy(acc_v[b], out_slice(i - NBUF),
                                  sem_o[b]).wait()

        pltpu.make_async_copy(xt.at[idx_all.at[pl.ds(i * G * K, G * K)]],
                              rows_v[b], sem_g[b]).wait()

        def node(g, carry):
            for m in range(C_IN // (2 * LANES)):
                sl = pl.ds(m * LANES, LANES)

                def row(k):
                    w = rows_v[b][g * K + k, sl]
                    return plsc.unpack(plsc.bitcast(w, jnp.bfloat16),
                                       format=plsc.PackFormat.INTERLEAVED)

                pa, pb = row(0)
                for k in range(1, K):
                    qa, qb = row(k)
                    pa = pa + qa
                    pb = pb + qb
                acc_v[b][g, pl.ds(m * LANES, LANES)] = pa
                acc_v[b][g, pl.ds(C_IN // 2 + m * LANES, LANES)] = pb
            return carry

        lax.fori_loop(0, G, node, 0)
        pltpu.async_copy(acc_v[b], out_slice(i), sem_o[b])

        @pl.when(i + NBUF < NBLK)
        def _():
            start_gather(i + NBUF, b)

    def blk(j, carry):
        for b in range(NBUF):
            do_block(j * NBUF + b, b)
        return carry

    lax.fori_loop(0, NBLK // NBUF, blk, 0)
    for b in range(NBUF):
        pltpu.make_async_copy(acc_v[b], out_slice(NBLK - NBUF + b),
                              sem_o[b]).wait()


_sc_gather = functools.partial(
    pl.kernel,
    mesh=plsc.VectorSubcoreMesh(core_axis_name="c", subcore_axis_name="s"),
    compiler_params=pltpu.CompilerParams(needs_layout_passes=False),
    out_type=jax.ShapeDtypeStruct((ROWS, C_IN), jnp.float32),
    scratch_types=[
        pltpu.VMEM((RPW * K,), jnp.int32),
        pltpu.VMEM((G * K, C_IN // 2), jnp.int32),
        pltpu.VMEM((G * K, C_IN // 2), jnp.int32),
        pltpu.VMEM((G, C_IN), jnp.float32),
        pltpu.VMEM((G, C_IN), jnp.float32),
        pltpu.SemaphoreType.DMA,
        pltpu.SemaphoreType.DMA,
        pltpu.SemaphoreType.DMA,
        pltpu.SemaphoreType.DMA,
    ],
)(_sc_body)


NBT = 2048


def _tr_body(x_ref, o_ref):
    xb = x_ref[0].T.astype(jnp.bfloat16)
    lo = lax.bitcast_convert_type(xb[:, :C_IN // 2], jnp.uint16)
    hi = lax.bitcast_convert_type(xb[:, C_IN // 2:], jnp.uint16)
    word = lo.astype(jnp.uint32) | (hi.astype(jnp.uint32) << 16)
    o_ref[...] = lax.bitcast_convert_type(word, jnp.int32)


def _tc_transpose(x3):
    return pl.pallas_call(
        _tr_body,
        grid=(N // NBT, B),
        in_specs=[pl.BlockSpec((1, C_IN, NBT), lambda i, b: (b, 0, i))],
        out_specs=pl.BlockSpec((NBT, C_IN // 2),
                               lambda i, b: (b * (N // NBT) + i, 0)),
        out_shape=jax.ShapeDtypeStruct((ROWS, C_IN // 2), jnp.int32),
    )(x3)


def _tc_body(x_ref, xj_ref, we_ref, w_ref, b_ref, o_ref):
    xb = x_ref[0]
    hj = xj_ref[...]
    s1 = lax.dot_general(we_ref[...], xb, (((1,), (0,)), ((), ())),
                         preferred_element_type=jnp.float32)
    s2 = lax.dot_general(w_ref[...], hj, (((1,), (1,)), ((), ())),
                         preferred_element_type=jnp.float32)
    o_ref[...] = jnp.maximum(s1 + s2 + b_ref[...], 0.0)[None]


def _tc_conv(x3, xj, W_eps, W, b2):
    nblk = N // NB
    return pl.pallas_call(
        _tc_body,
        grid=(ROWS // NB,),
        in_specs=[
            pl.BlockSpec((1, C_IN, NB), lambda i: (i // nblk, 0, i % nblk)),
            pl.BlockSpec((NB, C_IN), lambda i: (i, 0)),
            pl.BlockSpec((C_OUT, C_IN), lambda i: (0, 0)),
            pl.BlockSpec((C_OUT, C_IN), lambda i: (0, 0)),
            pl.BlockSpec((C_OUT, 1), lambda i: (0, 0)),
        ],
        out_specs=pl.BlockSpec((1, C_OUT, NB),
                               lambda i: (i // nblk, 0, i % nblk)),
        out_shape=jax.ShapeDtypeStruct((B, C_OUT, N), jnp.float32),
    )(x3, xj, W_eps, W, b2)


def kernel(x, edge_index, W, bconv, eps):
    xt = _tc_transpose(x.reshape(B, C_IN, N))
    idxg = edge_index[0].reshape(-1)
    xj = _sc_gather(xt, idxg)
    W_eps = (1.0 + eps[0]) * W
    out = _tc_conv(x.reshape(B, C_IN, N), xj, W_eps, W,
                   bconv.reshape(C_OUT, 1))
    return out.reshape(B, C_OUT, N, 1)


# speedup vs baseline: 6.2381x; 1.0114x over previous
import functools

import jax
import jax.numpy as jnp
from jax import lax
from jax.experimental import pallas as pl
from jax.experimental.pallas import tpu as pltpu
from jax.experimental.pallas import tpu_sc as plsc

B, C_IN, C_OUT, N, K = 4, 256, 256, 4096, 16
ROWS = B * N
NW = 32
RPW = ROWS // NW
G = 8
NBLK = RPW // G
NBUF = 2
LANES = 16
NB = 1024


def _sc_body(xt, idxg, xj, idx_all, rows0, rows1, acc0, acc1,
             sg0, sg1, so0, so1):
    rows_v = (rows0, rows1)
    acc_v = (acc0, acc1)
    sem_g = (sg0, sg1)
    sem_o = (so0, so1)
    wid = lax.axis_index("s") * 2 + lax.axis_index("c")
    base = wid * RPW

    pltpu.sync_copy(idxg.at[pl.ds(base * K, RPW * K)], idx_all)

    boff = lax.broadcast((wid // (NW // B)) * N, (LANES,))

    def rebase(i, carry):
        sl = pl.ds(i * LANES, LANES)
        idx_all[sl] = idx_all[sl] + boff
        return carry

    lax.fori_loop(0, RPW * K // LANES, rebase, 0)

    def start_gather(i, b):
        pltpu.async_copy(
            xt.at[idx_all.at[pl.ds(i * G * K, G * K)]], rows_v[b], sem_g[b])

    def out_slice(i):
        return xj.at[pl.ds(base + i * G, G)]

    for b in range(NBUF):
        start_gather(b, b)

    def do_block(i, b):
        @pl.when(i >= NBUF)
        def _():
            pltpu.make_async_copy(acc_v[b], out_slice(i - NBUF),
                                  sem_o[b]).wait()

        pltpu.make_async_copy(xt.at[idx_all.at[pl.ds(i * G * K, G * K)]],
                              rows_v[b], sem_g[b]).wait()

        def node(g, carry):
            for m in range(C_IN // (2 * LANES)):
                sl = pl.ds(m * LANES, LANES)

                def row(k):
                    w = rows_v[b][g * K + k, sl]
                    return plsc.unpack(plsc.bitcast(w, jnp.bfloat16),
                                       format=plsc.PackFormat.INTERLEAVED)

                pa, pb = row(0)
                for k in range(1, K):
                    qa, qb = row(k)
                    pa = pa + qa
                    pb = pb + qb
                acc_v[b][g, pl.ds(m * 2 * LANES, 2 * LANES)] = plsc.pack(
                    pa, pb, format=plsc.PackFormat.INTERLEAVED)
            return carry

        lax.fori_loop(0, G, node, 0)
        pltpu.async_copy(acc_v[b], out_slice(i), sem_o[b])

        @pl.when(i + NBUF < NBLK)
        def _():
            start_gather(i + NBUF, b)

    def blk(j, carry):
        for b in range(NBUF):
            do_block(j * NBUF + b, b)
        return carry

    lax.fori_loop(0, NBLK // NBUF, blk, 0)
    for b in range(NBUF):
        pltpu.make_async_copy(acc_v[b], out_slice(NBLK - NBUF + b),
                              sem_o[b]).wait()


_sc_gather = functools.partial(
    pl.kernel,
    mesh=plsc.VectorSubcoreMesh(core_axis_name="c", subcore_axis_name="s"),
    compiler_params=pltpu.CompilerParams(needs_layout_passes=False),
    out_type=jax.ShapeDtypeStruct((ROWS, C_IN), jnp.bfloat16),
    scratch_types=[
        pltpu.VMEM((RPW * K,), jnp.int32),
        pltpu.VMEM((G * K, C_IN // 2), jnp.int32),
        pltpu.VMEM((G * K, C_IN // 2), jnp.int32),
        pltpu.VMEM((G, C_IN), jnp.bfloat16),
        pltpu.VMEM((G, C_IN), jnp.bfloat16),
        pltpu.SemaphoreType.DMA,
        pltpu.SemaphoreType.DMA,
        pltpu.SemaphoreType.DMA,
        pltpu.SemaphoreType.DMA,
    ],
)(_sc_body)


NBT = 2048


def _tr_body(x_ref, o_ref):
    xb = x_ref[0].T.astype(jnp.bfloat16)
    lo = lax.bitcast_convert_type(xb[:, :C_IN // 2], jnp.uint16)
    hi = lax.bitcast_convert_type(xb[:, C_IN // 2:], jnp.uint16)
    word = lo.astype(jnp.uint32) | (hi.astype(jnp.uint32) << 16)
    o_ref[...] = lax.bitcast_convert_type(word, jnp.int32)


def _tc_transpose(x3):
    return pl.pallas_call(
        _tr_body,
        grid=(N // NBT, B),
        in_specs=[pl.BlockSpec((1, C_IN, NBT), lambda i, b: (b, 0, i))],
        out_specs=pl.BlockSpec((NBT, C_IN // 2),
                               lambda i, b: (b * (N // NBT) + i, 0)),
        out_shape=jax.ShapeDtypeStruct((ROWS, C_IN // 2), jnp.int32),
    )(x3)


def _tc_body(x_ref, xj_ref, we_ref, w_ref, b_ref, o_ref):
    xb = x_ref[0]
    hj = xj_ref[...]
    s1 = lax.dot_general(we_ref[...], xb, (((1,), (0,)), ((), ())),
                         preferred_element_type=jnp.float32)
    s2 = lax.dot_general(w_ref[...], hj, (((1,), (1,)), ((), ())),
                         preferred_element_type=jnp.float32)
    o_ref[...] = jnp.maximum(s1 + s2 + b_ref[...], 0.0)[None]


def _tc_conv(x3, xj, W_eps, W, b2):
    nblk = N // NB
    return pl.pallas_call(
        _tc_body,
        grid=(ROWS // NB,),
        in_specs=[
            pl.BlockSpec((1, C_IN, NB), lambda i: (i // nblk, 0, i % nblk)),
            pl.BlockSpec((NB, C_IN), lambda i: (i, 0)),
            pl.BlockSpec((C_OUT, C_IN), lambda i: (0, 0)),
            pl.BlockSpec((C_OUT, C_IN), lambda i: (0, 0)),
            pl.BlockSpec((C_OUT, 1), lambda i: (0, 0)),
        ],
        out_specs=pl.BlockSpec((1, C_OUT, NB),
                               lambda i: (i // nblk, 0, i % nblk)),
        out_shape=jax.ShapeDtypeStruct((B, C_OUT, N), jnp.float32),
    )(x3, xj, W_eps, W, b2)


def kernel(x, edge_index, W, bconv, eps):
    xt = _tc_transpose(x.reshape(B, C_IN, N))
    idxg = edge_index[0].reshape(-1)
    xj = _sc_gather(xt, idxg)
    W_eps = (1.0 + eps[0]) * W
    cols = jnp.arange(C_IN, dtype=jnp.int32)
    m32, r = cols // 32, cols % 32
    perm = jnp.where(r % 2 == 0, LANES * m32 + r // 2,
                     C_IN // 2 + LANES * m32 + r // 2)
    W_perm = W[:, perm].astype(jnp.bfloat16)
    out = _tc_conv(x.reshape(B, C_IN, N), xj, W_eps, W_perm,
                   bconv.reshape(C_OUT, 1))
    return out.reshape(B, C_OUT, N, 1)
